# Initial kernel scaffold; baseline (speedup 1.0000x reference)
import jax
import jax.numpy as jnp
from jax.experimental import pallas as pl


def kernel(z, table):
    raise NotImplementedError("write your pallas kernel here")


# speedup vs baseline: 1.1988x; 1.1988x over previous
import functools

import jax
import jax.numpy as jnp
from jax.experimental import pallas as pl
from jax.experimental.pallas import tpu as pltpu

B = 64
HW = 1024
N_CODES = 512
CODE_DIM = 32
N_VECS = B * HW


def _vq_kernel(z_ref, t_ref, zq_ref, idx_ref, loss_ref, counts_ref):
    step = pl.program_id(0)
    zb = z_ref[0]
    t = t_ref[...]

    mm = jax.lax.dot_general(
        zb, t, (((1,), (1,)), ((), ())),
        preferred_element_type=jnp.float32)
    zsq = jnp.sum(zb * zb, axis=-1, keepdims=True)
    tsq = jnp.sum(t * t, axis=-1)
    d = zsq + tsq[None, :] - 2.0 * mm

    idx = jnp.argmin(d, axis=-1).astype(jnp.int32)
    idx_ref[0, 0, :] = idx

    onehot = (jax.lax.broadcasted_iota(jnp.int32, (HW, N_CODES), 1)
              == idx[:, None]).astype(jnp.float32)
    zq = jax.lax.dot_general(
        onehot, t, (((1,), (0,)), ((), ())),
        precision=jax.lax.Precision.HIGHEST,
        preferred_element_type=jnp.float32)
    zq_ref[0] = zq

    diff = zq - zb
    part = jnp.sum(diff * diff)

    @pl.when(step == 0)
    def _init():
        loss_ref[...] = part.reshape(1, 1)
        counts_ref[...] = onehot

    @pl.when(step > 0)
    def _acc():
        loss_ref[...] += part.reshape(1, 1)
        counts_ref[...] += onehot


@jax.jit
def kernel(z, table):
    zq, idx3, loss, counts = pl.pallas_call(
        _vq_kernel,
        grid=(B,),
        in_specs=[
            pl.BlockSpec((1, HW, CODE_DIM), lambda b: (b, 0, 0)),
            pl.BlockSpec((N_CODES, CODE_DIM), lambda b: (0, 0)),
        ],
        out_specs=[
            pl.BlockSpec((1, HW, CODE_DIM), lambda b: (b, 0, 0)),
            pl.BlockSpec((1, 1, HW), lambda b: (b, 0, 0)),
            pl.BlockSpec((1, 1), lambda b: (0, 0)),
            pl.BlockSpec((HW, N_CODES), lambda b: (0, 0)),
        ],
        out_shape=[
            jax.ShapeDtypeStruct((B, HW, CODE_DIM), jnp.float32),
            jax.ShapeDtypeStruct((B, 1, HW), jnp.int32),
            jax.ShapeDtypeStruct((1, 1), jnp.float32),
            jax.ShapeDtypeStruct((HW, N_CODES), jnp.float32),
        ],
        compiler_params=pltpu.CompilerParams(
            dimension_semantics=("arbitrary",),
        ),
    )(z, table)

    total_loss = loss[0, 0] * (1.5 / (N_VECS * CODE_DIM))
    avg_probs = counts / jnp.float32(N_VECS)
    avg_probs = jnp.clip(avg_probs, 1e-10, None)
    avg_probs = avg_probs / avg_probs.sum()
    perplexity = jnp.exp(-jnp.sum(avg_probs * jnp.log(avg_probs)))
    return (zq, idx3.reshape(B, HW), total_loss, perplexity)


# speedup vs baseline: 2.4713x; 2.0615x over previous
import jax
import jax.numpy as jnp
from jax.experimental import pallas as pl
from jax.experimental.pallas import tpu as pltpu

B = 64
HW = 1024
N_CODES = 512
CODE_DIM = 32
N_VECS = B * HW


def _vq_kernel(z_ref, t_ref, tsq_ref, zq_ref, idx_ref, loss_ref, counts_ref):
    step = pl.program_id(0)
    zb = z_ref[0]
    t = t_ref[...]
    tsq = tsq_ref[...]

    mmT = jax.lax.dot_general(
        t, zb, (((1,), (1,)), ((), ())),
        preferred_element_type=jnp.float32)
    dT = tsq - 2.0 * mmT

    mn = jnp.min(dT, axis=0)
    idx = jnp.argmin(dT, axis=0).astype(jnp.int32)
    idx_ref[0, 0, :] = idx

    onehotT = (jax.lax.broadcasted_iota(jnp.int32, (N_CODES, HW), 0)
               == idx[None, :]).astype(jnp.float32)
    zq = jax.lax.dot_general(
        onehotT, t, (((0,), (0,)), ((), ())),
        preferred_element_type=jnp.float32)
    zq_ref[0] = zq

    part = jnp.sum(zb * zb) + jnp.sum(mn)

    @pl.when(step == 0)
    def _init():
        loss_ref[...] = part.reshape(1, 1)
        counts_ref[...] = onehotT

    @pl.when(step > 0)
    def _acc():
        loss_ref[...] += part.reshape(1, 1)
        counts_ref[...] += onehotT


@jax.jit
def kernel(z, table):
    tsq = jnp.sum(table * table, axis=-1, keepdims=True)
    zq, idx3, loss, countsT = pl.pallas_call(
        _vq_kernel,
        grid=(B,),
        in_specs=[
            pl.BlockSpec((1, HW, CODE_DIM), lambda b: (b, 0, 0)),
            pl.BlockSpec((N_CODES, CODE_DIM), lambda b: (0, 0)),
            pl.BlockSpec((N_CODES, 1), lambda b: (0, 0)),
        ],
        out_specs=[
            pl.BlockSpec((1, HW, CODE_DIM), lambda b: (b, 0, 0)),
            pl.BlockSpec((1, 1, HW), lambda b: (b, 0, 0)),
            pl.BlockSpec((1, 1), lambda b: (0, 0)),
            pl.BlockSpec((N_CODES, HW), lambda b: (0, 0)),
        ],
        out_shape=[
            jax.ShapeDtypeStruct((B, HW, CODE_DIM), jnp.float32),
            jax.ShapeDtypeStruct((B, 1, HW), jnp.int32),
            jax.ShapeDtypeStruct((1, 1), jnp.float32),
            jax.ShapeDtypeStruct((N_CODES, HW), jnp.float32),
        ],
        compiler_params=pltpu.CompilerParams(
            dimension_semantics=("arbitrary",),
        ),
    )(z, table, tsq)

    total_loss = loss[0, 0] * (1.5 / (N_VECS * CODE_DIM))
    avg_probs = countsT / jnp.float32(N_VECS)
    avg_probs = jnp.clip(avg_probs, 1e-10, None)
    avg_probs = avg_probs / avg_probs.sum()
    perplexity = jnp.exp(-jnp.sum(avg_probs * jnp.log(avg_probs)))
    return (zq, idx3.reshape(B, HW), total_loss, perplexity)


# speedup vs baseline: 2.7678x; 1.1200x over previous
import jax
import jax.numpy as jnp
from jax.experimental import pallas as pl
from jax.experimental.pallas import tpu as pltpu

B = 64
HW = 1024
N_CODES = 512
CODE_DIM = 32
N_VECS = B * HW
R = 2
RW = R * HW


def _vq_kernel(z_ref, t_ref, tsq_ref, zq_ref, idx_ref, loss_ref, counts_ref):
    step = pl.program_id(0)
    zb = z_ref[...].reshape(RW, CODE_DIM)
    t = t_ref[...]
    tsq = tsq_ref[...]

    mmT = jax.lax.dot_general(
        t, zb, (((1,), (1,)), ((), ())),
        preferred_element_type=jnp.float32)
    dT = tsq - 2.0 * mmT

    mn = jnp.min(dT, axis=0)
    idx = jnp.argmin(dT, axis=0).astype(jnp.int32)
    idx_ref[0, 0, :] = idx

    onehotT = (jax.lax.broadcasted_iota(jnp.int32, (N_CODES, RW), 0)
               == idx[None, :]).astype(jnp.float32)
    zq = jax.lax.dot_general(
        onehotT, t, (((0,), (0,)), ((), ())),
        preferred_element_type=jnp.float32)
    zq_ref[...] = zq.reshape(R, HW, CODE_DIM)

    hot = onehotT[:, 0:HW]
    for r in range(1, R):
        hot = hot + onehotT[:, r * HW:(r + 1) * HW]

    part = jnp.sum(zb * zb) + jnp.sum(mn)

    first = step == 0
    loss_ref[...] = jnp.where(first, 0.0, loss_ref[...]) + part.reshape(1, 1)
    counts_ref[...] = jnp.where(first, 0.0, counts_ref[...]) + hot


@jax.jit
def kernel(z, table):
    tsq = jnp.sum(table * table, axis=-1, keepdims=True)
    zq, idx3, loss, countsT = pl.pallas_call(
        _vq_kernel,
        grid=(B // R,),
        in_specs=[
            pl.BlockSpec((R, HW, CODE_DIM), lambda b: (b, 0, 0)),
            pl.BlockSpec((N_CODES, CODE_DIM), lambda b: (0, 0)),
            pl.BlockSpec((N_CODES, 1), lambda b: (0, 0)),
        ],
        out_specs=[
            pl.BlockSpec((R, HW, CODE_DIM), lambda b: (b, 0, 0)),
            pl.BlockSpec((1, 1, RW), lambda b: (b, 0, 0)),
            pl.BlockSpec((1, 1), lambda b: (0, 0)),
            pl.BlockSpec((N_CODES, HW), lambda b: (0, 0)),
        ],
        out_shape=[
            jax.ShapeDtypeStruct((B, HW, CODE_DIM), jnp.float32),
            jax.ShapeDtypeStruct((B // R, 1, RW), jnp.int32),
            jax.ShapeDtypeStruct((1, 1), jnp.float32),
            jax.ShapeDtypeStruct((N_CODES, HW), jnp.float32),
        ],
        compiler_params=pltpu.CompilerParams(
            dimension_semantics=("arbitrary",),
        ),
    )(z, table, tsq)

    total_loss = loss[0, 0] * (1.5 / (N_VECS * CODE_DIM))
    avg_probs = countsT / jnp.float32(N_VECS)
    avg_probs = jnp.clip(avg_probs, 1e-10, None)
    avg_probs = avg_probs / avg_probs.sum()
    perplexity = jnp.exp(-jnp.sum(avg_probs * jnp.log(avg_probs)))
    return (zq, idx3.reshape(B, HW), total_loss, perplexity)
